# baseline (speedup 1.0000x reference)
import functools

import jax
import jax.numpy as jnp
from jax import lax
from jax.experimental import pallas as pl
from jax.experimental.pallas import tpu as pltpu

_VMEM_LIMIT = 60000 * 1024
_BF = jnp.bfloat16


def _bias_f32(b_ref):
    return b_ref[...].astype(jnp.float32).reshape(1, 1, -1)


def _lateral_into(x_ref, w_ref, b_ref, lat_ref, *, H, chunk):
    bias = _bias_f32(b_ref)
    for c0 in range(0, H, chunk):
        y = lax.dot_general(x_ref[0, c0:c0 + chunk], w_ref[...],
                            dimension_numbers=(((2,), (0,)), ((), ())),
                            preferred_element_type=jnp.float32)
        lat_ref[0, c0:c0 + chunk] = (y + bias).astype(_BF)


def _upsample_add_into(x_ref, w_ref, b_ref, r_ref, lat_ref, *, H, W, chunk):
    bias = _bias_f32(b_ref)
    F = w_ref.shape[1]
    W2 = W // 2
    wf = lax.broadcasted_iota(jnp.int32, (W, W2), 0)
    wc = lax.broadcasted_iota(jnp.int32, (W, W2), 1)
    up = jnp.broadcast_to(((wc == wf // 2).astype(_BF))[None],
                          (chunk // 2, W, W2))
    for c0 in range(0, H, chunk):
        y = lax.dot_general(x_ref[0, c0:c0 + chunk], w_ref[...],
                            dimension_numbers=(((2,), (0,)), ((), ())),
                            preferred_element_type=jnp.float32)
        r = r_ref[0, c0 // 2:(c0 + chunk) // 2]
        r_up = lax.dot_general(up, r,
                               dimension_numbers=(((2,), (1,)), ((0,), (0,))),
                               preferred_element_type=jnp.float32)
        out = (y.reshape(chunk // 2, 2, W, F) + bias[None]
               + r_up[:, None, :, :])
        lat_ref[0, c0:c0 + chunk] = out.reshape(chunk, W, F).astype(_BF)


def _eye_bf(F):
    a = lax.broadcasted_iota(jnp.int32, (F, F), 0)
    b = lax.broadcasted_iota(jnp.int32, (F, F), 1)
    return (a == b).astype(_BF)


def _to_nchw(t_bf, F):
    return lax.dot_general(_eye_bf(F), t_bf,
                           dimension_numbers=(((1,), (2,)), ((), ())),
                           preferred_element_type=jnp.float32)


def _conv3x3_into(lat_ref, w_ref, b_ref, o_ref, acc_ref, *, H, W, chunk):
    bias = _bias_f32(b_ref)
    F = w_ref.shape[2]

    def tap(lhs, k):
        return lax.dot_general(lhs, w_ref[k],
                               dimension_numbers=(((2,), (0,)), ((), ())),
                               preferred_element_type=jnp.float32)

    for c0 in range(0, H, chunk):
        acc_ref[...] = jnp.broadcast_to(bias, (chunk, W, F))
        for dy in range(3):
            d_lo = max(0, 1 - dy - c0)
            d_hi = min(chunk, H - c0 - dy + 1)
            rows = lat_ref[0, c0 + d_lo + dy - 1:c0 + d_hi + dy - 1]
            acc_ref[d_lo:d_hi] += tap(rows, 3 * dy + 1)
            acc_ref[d_lo:d_hi, 1:W] += tap(rows[:, 0:W - 1], 3 * dy + 0)
            acc_ref[d_lo:d_hi, 0:W - 1] += tap(rows[:, 1:W], 3 * dy + 2)
        o_ref[0, :, c0:c0 + chunk] = _to_nchw(acc_ref[...].astype(_BF), F)


def _p54_kernel(c4_ref, c5_ref,
                w5l_ref, b5l_ref, w4l_ref, b4l_ref,
                w5s_ref, b5s_ref, w4s_ref, b4s_ref,
                o5_ref, o4_ref, olat4_ref,
                lat5_ref, acc5_ref, acc4_ref, *, H5, W5, H4, W4):
    _lateral_into(c5_ref, w5l_ref, b5l_ref, lat5_ref, H=H5, chunk=H5)
    _conv3x3_into(lat5_ref, w5s_ref, b5s_ref, o5_ref, acc5_ref,
                  H=H5, W=W5, chunk=H5)
    _upsample_add_into(c4_ref, w4l_ref, b4l_ref, lat5_ref, olat4_ref,
                       H=H4, W=W4, chunk=H4 // 2)
    _conv3x3_into(olat4_ref, w4s_ref, b4s_ref, o4_ref, acc4_ref,
                  H=H4, W=W4, chunk=H4 // 2)


def _p3_kernel(c3_ref, lat4_ref, w3l_ref, b3l_ref, w3s_ref, b3s_ref,
               o3_ref, lat3_ref, acc3_ref, *, H3, W3, chunk3):
    _upsample_add_into(c3_ref, w3l_ref, b3l_ref, lat4_ref, lat3_ref,
                       H=H3, W=W3, chunk=H3 // 4)
    _conv3x3_into(lat3_ref, w3s_ref, b3s_ref, o3_ref, acc3_ref,
                  H=H3, W=W3, chunk=chunk3)


def _s2_conv_from(x, w_ref, b_ref, acc_ref, *, H, W, F):
    H2 = H // 2
    Wo = (W - 1) // 2 + 1
    Cin = x.shape[-1]
    x5 = x.reshape(H2, 2, W, Cin)
    acc_ref[...] = jnp.broadcast_to(_bias_f32(b_ref), (H2, W, F))

    def tap(lhs, k):
        return lax.dot_general(lhs, w_ref[k],
                               dimension_numbers=(((2,), (0,)), ((), ())),
                               preferred_element_type=jnp.float32)

    def cols(rows, ky, ro, nr):
        acc_ref[ro:ro + nr] += tap(rows, 3 * ky + 1)
        acc_ref[ro:ro + nr, 1:W] += tap(rows[:, 0:W - 1], 3 * ky + 0)
        acc_ref[ro:ro + nr, 0:W - 1] += tap(rows[:, 1:W], 3 * ky + 2)

    cols(x5[:, 0], 1, 0, H2)
    cols(x5[:, 1], 2, 0, H2)
    if H2 > 1:
        cols(x5[0:H2 - 1, 1], 0, 1, H2 - 1)

    wo = lax.broadcasted_iota(jnp.int32, (Wo, W), 0)
    wi = lax.broadcasted_iota(jnp.int32, (Wo, W), 1)
    sel = jnp.broadcast_to(((wi == 2 * wo).astype(jnp.float32))[None],
                           (H2, Wo, W))
    return lax.dot_general(sel, acc_ref[...],
                           dimension_numbers=(((2,), (1,)), ((0,), (0,))),
                           preferred_element_type=jnp.float32)


def _p67_kernel(c5_ref, w6_ref, b6_ref, w7_ref, b7_ref, o6_ref, o7_ref,
                acc6_ref, acc7_ref, *, H5, W5, F):
    p6 = _s2_conv_from(c5_ref[0], w6_ref, b6_ref, acc6_ref,
                       H=H5, W=W5, F=F)
    p6b = p6.astype(_BF)
    o6_ref[0] = _to_nchw(p6b, F)
    p6r = jnp.maximum(p6b, 0)
    H6 = H5 // 2
    W6 = (W5 - 1) // 2 + 1
    p7 = _s2_conv_from(p6r, w7_ref, b7_ref, acc7_ref, H=H6, W=W6, F=F)
    o7_ref[0] = _to_nchw(p7.astype(_BF), F)


def kernel(C3, C4, C5, P5_1_w, P5_1_b, P5_2_w, P5_2_b, P4_1_w, P4_1_b,
           P4_2_w, P4_2_b, P3_1_w, P3_1_b, P3_2_w, P3_2_b, P6_w, P6_b,
           P7_2_w, P7_2_b):
    N, C3c, H3, W3 = C3.shape
    _, C4c, H4, W4 = C4.shape
    _, C5c, H5, W5 = C5.shape
    F = P5_1_w.shape[1]

    to_nhwc = lambda t: jnp.transpose(t.astype(_BF), (0, 2, 3, 1))
    c3 = to_nhwc(C3)
    c4 = to_nhwc(C4)
    c5 = to_nhwc(C5)
    bf = lambda w: w.astype(_BF)
    b2 = lambda b: b.reshape(1, F)

    full = lambda *shape: pl.BlockSpec(shape, lambda n: (0,) * len(shape))
    img = lambda H, W, C: pl.BlockSpec((1, H, W, C), lambda n: (n, 0, 0, 0))
    nchw = lambda H, W: jax.ShapeDtypeStruct((N, F, H, W), jnp.float32)
    cp = pltpu.CompilerParams(dimension_semantics=("parallel",),
                              vmem_limit_bytes=_VMEM_LIMIT)

    body_a = functools.partial(_p54_kernel, H5=H5, W5=W5, H4=H4, W4=W4)
    o5, o4, lat4 = pl.pallas_call(
        body_a,
        out_shape=[nchw(H5, W5), nchw(H4, W4),
                   jax.ShapeDtypeStruct((N, H4, W4, F), _BF)],
        grid=(N,),
        in_specs=[
            img(H4, W4, C4c), img(H5, W5, C5c),
            full(C5c, F), full(1, F), full(C4c, F), full(1, F),
            full(9, F, F), full(1, F), full(9, F, F), full(1, F),
        ],
        out_specs=[img(F, H5, W5), img(F, H4, W4), img(H4, W4, F)],
        scratch_shapes=[
            pltpu.VMEM((1, H5, W5, F), _BF),
            pltpu.VMEM((H5, W5, F), jnp.float32),
            pltpu.VMEM((H4 // 2, W4, F), jnp.float32),
        ],
        compiler_params=cp,
    )(c4, c5, bf(P5_1_w), b2(P5_1_b), bf(P4_1_w), b2(P4_1_b),
      bf(P5_2_w), b2(P5_2_b), bf(P4_2_w), b2(P4_2_b))

    chunk3 = H3 // 8 if H3 % 8 == 0 else H3
    body_b = functools.partial(_p3_kernel, H3=H3, W3=W3, chunk3=chunk3)
    o3 = pl.pallas_call(
        body_b,
        out_shape=nchw(H3, W3),
        grid=(N,),
        in_specs=[
            img(H3, W3, C3c), img(H4, W4, F),
            full(C3c, F), full(1, F), full(9, F, F), full(1, F),
        ],
        out_specs=img(F, H3, W3),
        scratch_shapes=[
            pltpu.VMEM((1, H3, W3, F), _BF),
            pltpu.VMEM((chunk3, W3, F), jnp.float32),
        ],
        compiler_params=cp,
    )(c3, lat4, bf(P3_1_w), b2(P3_1_b), bf(P3_2_w), b2(P3_2_b))

    H6, W6 = H5 // 2, (W5 - 1) // 2 + 1
    H7, W7 = H6 // 2, (W6 - 1) // 2 + 1
    body_c = functools.partial(_p67_kernel, H5=H5, W5=W5, F=F)
    o6, o7 = pl.pallas_call(
        body_c,
        out_shape=[nchw(H6, W6), nchw(H7, W7)],
        grid=(N,),
        in_specs=[
            img(H5, W5, C5c),
            full(9, C5c, F), full(1, F), full(9, F, F), full(1, F),
        ],
        out_specs=[img(F, H6, W6), img(F, H7, W7)],
        scratch_shapes=[
            pltpu.VMEM((H5 // 2, W5, F), jnp.float32),
            pltpu.VMEM((H6 // 2, W6, F), jnp.float32),
        ],
        compiler_params=cp,
    )(c5, bf(P6_w), b2(P6_b), bf(P7_2_w), b2(P7_2_b))

    return [o3, o4, o5, o6, o7]


# speedup vs baseline: 1.1919x; 1.1919x over previous
import functools

import jax
import jax.numpy as jnp
from jax import lax
from jax.experimental import pallas as pl
from jax.experimental.pallas import tpu as pltpu

_VMEM_LIMIT = 48 * 1024 * 1024
_BF = jnp.bfloat16


def _pw_kernel(x_ref, w_ref, b_ref, o_ref):
    y = jnp.dot(x_ref[...], w_ref[...], preferred_element_type=jnp.float32)
    o_ref[...] = (y + b_ref[...].astype(jnp.float32)).astype(o_ref.dtype)


def _conv1x1(x, w, b):
    N, H, W, Cin = x.shape
    F = w.shape[1]
    M = N * H * W
    xf = x.reshape(M, Cin)
    TM = M // 2 if M % 2 == 0 else M
    grid = (M // TM,)
    out = pl.pallas_call(
        _pw_kernel,
        out_shape=jax.ShapeDtypeStruct((M, F), _BF),
        grid=grid,
        in_specs=[
            pl.BlockSpec((TM, Cin), lambda m: (m, 0)),
            pl.BlockSpec((Cin, F), lambda m: (0, 0)),
            pl.BlockSpec((1, F), lambda m: (0, 0)),
        ],
        out_specs=pl.BlockSpec((TM, F), lambda m: (m, 0)),
        compiler_params=pltpu.CompilerParams(
            dimension_semantics=("parallel",),
            vmem_limit_bytes=_VMEM_LIMIT),
    )(xf, w, b.reshape(1, F))
    return out.reshape(N, H, W, F)


def _pw_upadd_kernel(x_ref, w_ref, b_ref, r_ref, o_ref):
    _, TH2, _, W, Cin = x_ref.shape
    F = w_ref.shape[1]
    W2 = r_ref.shape[2]

    xa = x_ref[0].reshape(TH2 * 2, W, Cin)
    ya = lax.dot_general(xa, w_ref[...],
                         dimension_numbers=(((2,), (0,)), ((), ())),
                         preferred_element_type=jnp.float32)

    r = r_ref[0]
    wf = lax.broadcasted_iota(jnp.int32, (W, W2), 0)
    wc = lax.broadcasted_iota(jnp.int32, (W, W2), 1)
    up = (wc == wf // 2).astype(_BF)
    upb = jnp.broadcast_to(up[None], (TH2, W, W2))
    r_up = lax.dot_general(upb, r,
                           dimension_numbers=(((2,), (1,)), ((0,), (0,))),
                           preferred_element_type=jnp.float32)

    bias = b_ref[...].astype(jnp.float32).reshape(1, 1, 1, F)
    out = ya.reshape(TH2, 2, W, F) + bias + r_up[:, None, :, :]
    o_ref[0] = out.astype(o_ref.dtype)


def _conv1x1_upsample_add(x, w, b, r):
    N, H, W, Cin = x.shape
    F = w.shape[1]
    H2, W2 = H // 2, W // 2

    x5 = x.reshape(N, H2, 2, W, Cin)
    TH2 = H2 // 2 if H2 % 2 == 0 else H2
    grid = (N, H2 // TH2)

    out5 = pl.pallas_call(
        _pw_upadd_kernel,
        out_shape=jax.ShapeDtypeStruct((N, H2, 2, W, F), _BF),
        grid=grid,
        in_specs=[
            pl.BlockSpec((1, TH2, 2, W, Cin), lambda n, i: (n, i, 0, 0, 0)),
            pl.BlockSpec((Cin, F), lambda n, i: (0, 0)),
            pl.BlockSpec((1, F), lambda n, i: (0, 0)),
            pl.BlockSpec((1, TH2, W2, F), lambda n, i: (n, i, 0, 0)),
        ],
        out_specs=pl.BlockSpec((1, TH2, 2, W, F), lambda n, i: (n, i, 0, 0, 0)),
        compiler_params=pltpu.CompilerParams(
            dimension_semantics=("parallel", "parallel"),
            vmem_limit_bytes=_VMEM_LIMIT),
    )(x5, w, b.reshape(1, F), r.reshape(N, H2, W2, F))
    return out5.reshape(N, H, W, F)


def _c3s1_kernel(x_ref, w_ref, b_ref, o_ref, acc_ref, *, TH, W, Cout):
    i = pl.program_id(1)
    bias = b_ref[...].astype(jnp.float32).reshape(1, 1, Cout)
    acc_ref[...] = jnp.broadcast_to(bias, (TH, W, Cout))

    def tap(lhs, k):
        return lax.dot_general(lhs, w_ref[k],
                               dimension_numbers=(((2,), (0,)), ((), ())),
                               preferred_element_type=jnp.float32)

    for dy in range(3):
        rows = x_ref[0, pl.ds(i * TH + dy, TH)]
        acc_ref[...] += tap(rows, 3 * dy + 1)
        acc_ref[:, 1:W] += tap(rows[:, 0:W - 1], 3 * dy + 0)
        acc_ref[:, 0:W - 1] += tap(rows[:, 1:W], 3 * dy + 2)
    o_ref[0] = acc_ref[...].astype(o_ref.dtype)


def _conv3x3_s1(x, w9, b, row_tile):
    N, H, W, Cin = x.shape
    Cout = w9.shape[-1]
    TH = min(row_tile, H)
    xp = jnp.pad(x, ((0, 0), (1, 1), (0, 0), (0, 0)))
    body = functools.partial(_c3s1_kernel, TH=TH, W=W, Cout=Cout)
    return pl.pallas_call(
        body,
        out_shape=jax.ShapeDtypeStruct((N, H, W, Cout), _BF),
        grid=(N, H // TH),
        in_specs=[
            pl.BlockSpec((1, H + 2, W, Cin), lambda n, i: (n, 0, 0, 0)),
            pl.BlockSpec((9, Cin, Cout), lambda n, i: (0, 0, 0)),
            pl.BlockSpec((1, Cout), lambda n, i: (0, 0)),
        ],
        out_specs=pl.BlockSpec((1, TH, W, Cout), lambda n, i: (n, i, 0, 0)),
        scratch_shapes=[pltpu.VMEM((TH, W, Cout), jnp.float32)],
        compiler_params=pltpu.CompilerParams(
            dimension_semantics=("parallel", "arbitrary"),
            vmem_limit_bytes=_VMEM_LIMIT),
    )(xp, w9, b.reshape(1, Cout))


def _c3s2_kernel(x_ref, w_ref, b_ref, o_ref, acc_ref, *,
                 H2, W_in, W_out, Cout, apply_relu):
    x = x_ref[0]
    if apply_relu:
        x = jnp.maximum(x, jnp.zeros_like(x))
    Cin = x.shape[-1]
    x5 = x.reshape(H2, 2, W_in, Cin)

    bias = b_ref[...].astype(jnp.float32).reshape(1, 1, Cout)
    acc_ref[...] = jnp.broadcast_to(bias, (H2, W_in, Cout))

    def tap(lhs, k):
        return lax.dot_general(lhs, w_ref[k],
                               dimension_numbers=(((2,), (0,)), ((), ())),
                               preferred_element_type=jnp.float32)

    def cols(rows, ky, ro, nr):
        acc_ref[ro:ro + nr] += tap(rows, 3 * ky + 1)
        acc_ref[ro:ro + nr, 1:W_in] += tap(rows[:, 0:W_in - 1], 3 * ky + 0)
        acc_ref[ro:ro + nr, 0:W_in - 1] += tap(rows[:, 1:W_in], 3 * ky + 2)

    cols(x5[:, 0], 1, 0, H2)
    cols(x5[:, 1], 2, 0, H2)
    if H2 > 1:
        cols(x5[0:H2 - 1, 1], 0, 1, H2 - 1)

    wo = lax.broadcasted_iota(jnp.int32, (W_out, W_in), 0)
    wi = lax.broadcasted_iota(jnp.int32, (W_out, W_in), 1)
    sel = (wi == 2 * wo).astype(jnp.float32)
    selb = jnp.broadcast_to(sel[None], (H2, W_out, W_in))
    out = lax.dot_general(selb, acc_ref[...],
                          dimension_numbers=(((2,), (1,)), ((0,), (0,))),
                          preferred_element_type=jnp.float32)
    o_ref[0] = out.astype(o_ref.dtype)


def _conv3x3_s2(x, w9, b, apply_relu=False):
    N, H_in, W_in, Cin = x.shape
    Cout = w9.shape[-1]
    H2 = H_in // 2
    W_out = (W_in - 1) // 2 + 1
    body = functools.partial(_c3s2_kernel, H2=H2, W_in=W_in, W_out=W_out,
                             Cout=Cout, apply_relu=apply_relu)
    return pl.pallas_call(
        body,
        out_shape=jax.ShapeDtypeStruct((N, H2, W_out, Cout), _BF),
        grid=(N,),
        in_specs=[
            pl.BlockSpec((1, H_in, W_in, Cin), lambda n: (n, 0, 0, 0)),
            pl.BlockSpec((9, Cin, Cout), lambda n: (0, 0, 0)),
            pl.BlockSpec((1, Cout), lambda n: (0, 0)),
        ],
        out_specs=pl.BlockSpec((1, H2, W_out, Cout), lambda n: (n, 0, 0, 0)),
        scratch_shapes=[pltpu.VMEM((H2, W_in, Cout), jnp.float32)],
        compiler_params=pltpu.CompilerParams(
            dimension_semantics=("parallel",),
            vmem_limit_bytes=_VMEM_LIMIT),
    )(x, w9, b.reshape(1, Cout))


def kernel(C3, C4, C5, P5_1_w, P5_1_b, P5_2_w, P5_2_b, P4_1_w, P4_1_b,
           P4_2_w, P4_2_b, P3_1_w, P3_1_b, P3_2_w, P3_2_b, P6_w, P6_b,
           P7_2_w, P7_2_b):
    to_nhwc = lambda t: jnp.transpose(t.astype(_BF), (0, 2, 3, 1))
    c3 = to_nhwc(C3)
    c4 = to_nhwc(C4)
    c5 = to_nhwc(C5)

    p5_lat = _conv1x1(c5, P5_1_w.astype(_BF), P5_1_b)
    p5 = _conv3x3_s1(p5_lat, P5_2_w.astype(_BF), P5_2_b, 20)

    p4_lat = _conv1x1_upsample_add(c4, P4_1_w.astype(_BF), P4_1_b, p5_lat)
    p4 = _conv3x3_s1(p4_lat, P4_2_w.astype(_BF), P4_2_b, 20)

    p3_lat = _conv1x1_upsample_add(c3, P3_1_w.astype(_BF), P3_1_b, p4_lat)
    p3 = _conv3x3_s1(p3_lat, P3_2_w.astype(_BF), P3_2_b, 20)

    p6 = _conv3x3_s2(c5, P6_w.astype(_BF), P6_b)
    p7 = _conv3x3_s2(p6, P7_2_w.astype(_BF), P7_2_b, apply_relu=True)

    to_nchw = lambda t: jnp.transpose(t, (0, 3, 1, 2)).astype(jnp.float32)
    return [to_nchw(p3), to_nchw(p4), to_nchw(p5), to_nchw(p6), to_nchw(p7)]


# speedup vs baseline: 1.2774x; 1.0717x over previous
import functools

import jax
import jax.numpy as jnp
from jax import lax
from jax.experimental import pallas as pl
from jax.experimental.pallas import tpu as pltpu

_VMEM_LIMIT = 48 * 1024 * 1024
_BF = jnp.bfloat16


def _pw_kernel(x_ref, w_ref, b_ref, o_ref):
    y = jnp.dot(x_ref[...], w_ref[...], preferred_element_type=jnp.float32)
    o_ref[...] = (y + b_ref[...].astype(jnp.float32)).astype(o_ref.dtype)


def _conv1x1(x, w, b):
    N, H, W, Cin = x.shape
    F = w.shape[1]
    M = N * H * W
    xf = x.reshape(M, Cin)
    TM = M // 2 if M % 2 == 0 else M
    grid = (M // TM,)
    out = pl.pallas_call(
        _pw_kernel,
        out_shape=jax.ShapeDtypeStruct((M, F), jnp.float32),
        grid=grid,
        in_specs=[
            pl.BlockSpec((TM, Cin), lambda m: (m, 0)),
            pl.BlockSpec((Cin, F), lambda m: (0, 0)),
            pl.BlockSpec((1, F), lambda m: (0, 0)),
        ],
        out_specs=pl.BlockSpec((TM, F), lambda m: (m, 0)),
        compiler_params=pltpu.CompilerParams(
            dimension_semantics=("parallel",),
            vmem_limit_bytes=_VMEM_LIMIT),
    )(xf, w, b.reshape(1, F))
    return out.reshape(N, H, W, F)


def _pw_upadd_kernel(x_ref, w_ref, b_ref, r_ref, o_ref):
    _, TH2, _, W, Cin = x_ref.shape
    F = w_ref.shape[1]
    W2 = r_ref.shape[2]

    xa = x_ref[0].reshape(TH2 * 2, W, Cin)
    ya = lax.dot_general(xa, w_ref[...],
                         dimension_numbers=(((2,), (0,)), ((), ())),
                         preferred_element_type=jnp.float32)

    r = r_ref[0]
    wf = lax.broadcasted_iota(jnp.int32, (W, W2), 0)
    wc = lax.broadcasted_iota(jnp.int32, (W, W2), 1)
    up = (wc == wf // 2).astype(_BF)
    upb = jnp.broadcast_to(up[None], (TH2, W, W2))
    r_up = lax.dot_general(upb, r,
                           dimension_numbers=(((2,), (1,)), ((0,), (0,))),
                           preferred_element_type=jnp.float32)

    bias = b_ref[...].astype(jnp.float32).reshape(1, 1, 1, F)
    out = ya.reshape(TH2, 2, W, F) + bias + r_up[:, None, :, :]
    o_ref[0] = out.astype(o_ref.dtype)


def _conv1x1_upsample_add(x, w, b, r):
    N, H, W, Cin = x.shape
    F = w.shape[1]
    H2, W2 = H // 2, W // 2

    x5 = x.reshape(N, H2, 2, W, Cin)
    TH2 = H2 // 2 if H2 % 2 == 0 else H2
    grid = (N, H2 // TH2)

    out5 = pl.pallas_call(
        _pw_upadd_kernel,
        out_shape=jax.ShapeDtypeStruct((N, H2, 2, W, F), jnp.float32),
        grid=grid,
        in_specs=[
            pl.BlockSpec((1, TH2, 2, W, Cin), lambda n, i: (n, i, 0, 0, 0)),
            pl.BlockSpec((Cin, F), lambda n, i: (0, 0)),
            pl.BlockSpec((1, F), lambda n, i: (0, 0)),
            pl.BlockSpec((1, TH2, W2, F), lambda n, i: (n, i, 0, 0)),
        ],
        out_specs=pl.BlockSpec((1, TH2, 2, W, F), lambda n, i: (n, i, 0, 0, 0)),
        compiler_params=pltpu.CompilerParams(
            dimension_semantics=("parallel", "parallel"),
            vmem_limit_bytes=_VMEM_LIMIT),
    )(x5, w, b.reshape(1, F), r.reshape(N, H2, W2, F))
    return out5.reshape(N, H, W, F)


def _c3s1_kernel(x_ref, w_ref, b_ref, o_ref, acc_ref, *, TH, W, Cout):
    i = pl.program_id(1)
    bias = b_ref[...].astype(jnp.float32).reshape(1, 1, Cout)
    acc_ref[...] = jnp.broadcast_to(bias, (TH, W, Cout))

    def tap(lhs, k):
        return lax.dot_general(lhs, w_ref[k],
                               dimension_numbers=(((2,), (0,)), ((), ())),
                               preferred_element_type=jnp.float32)

    for dy in range(3):
        rows = x_ref[0, pl.ds(i * TH + dy, TH)]
        acc_ref[...] += tap(rows, 3 * dy + 1)
        acc_ref[:, 1:W] += tap(rows[:, 0:W - 1], 3 * dy + 0)
        acc_ref[:, 0:W - 1] += tap(rows[:, 1:W], 3 * dy + 2)
    o_ref[0] = acc_ref[...].astype(o_ref.dtype)


def _conv3x3_s1(x, w9, b, row_tile):
    N, H, W, Cin = x.shape
    Cout = w9.shape[-1]
    TH = min(row_tile, H)
    xp = jnp.pad(x, ((0, 0), (1, 1), (0, 0), (0, 0)))
    body = functools.partial(_c3s1_kernel, TH=TH, W=W, Cout=Cout)
    return pl.pallas_call(
        body,
        out_shape=jax.ShapeDtypeStruct((N, H, W, Cout), jnp.float32),
        grid=(N, H // TH),
        in_specs=[
            pl.BlockSpec((1, H + 2, W, Cin), lambda n, i: (n, 0, 0, 0)),
            pl.BlockSpec((9, Cin, Cout), lambda n, i: (0, 0, 0)),
            pl.BlockSpec((1, Cout), lambda n, i: (0, 0)),
        ],
        out_specs=pl.BlockSpec((1, TH, W, Cout), lambda n, i: (n, i, 0, 0)),
        scratch_shapes=[pltpu.VMEM((TH, W, Cout), jnp.float32)],
        compiler_params=pltpu.CompilerParams(
            dimension_semantics=("parallel", "arbitrary"),
            vmem_limit_bytes=_VMEM_LIMIT),
    )(xp, w9, b.reshape(1, Cout))


def _c3s2_kernel(x_ref, w_ref, b_ref, o_ref, acc_ref, *,
                 H2, W_in, W_out, Cout, apply_relu):
    x = x_ref[0]
    if apply_relu:
        x = jnp.maximum(x, jnp.zeros_like(x))
    Cin = x.shape[-1]
    x5 = x.reshape(H2, 2, W_in, Cin)

    bias = b_ref[...].astype(jnp.float32).reshape(1, 1, Cout)
    acc_ref[...] = jnp.broadcast_to(bias, (H2, W_in, Cout))

    def tap(lhs, k):
        return lax.dot_general(lhs, w_ref[k],
                               dimension_numbers=(((2,), (0,)), ((), ())),
                               preferred_element_type=jnp.float32)

    def cols(rows, ky, ro, nr):
        acc_ref[ro:ro + nr] += tap(rows, 3 * ky + 1)
        acc_ref[ro:ro + nr, 1:W_in] += tap(rows[:, 0:W_in - 1], 3 * ky + 0)
        acc_ref[ro:ro + nr, 0:W_in - 1] += tap(rows[:, 1:W_in], 3 * ky + 2)

    cols(x5[:, 0], 1, 0, H2)
    cols(x5[:, 1], 2, 0, H2)
    if H2 > 1:
        cols(x5[0:H2 - 1, 1], 0, 1, H2 - 1)

    wo = lax.broadcasted_iota(jnp.int32, (W_out, W_in), 0)
    wi = lax.broadcasted_iota(jnp.int32, (W_out, W_in), 1)
    sel = (wi == 2 * wo).astype(jnp.float32)
    selb = jnp.broadcast_to(sel[None], (H2, W_out, W_in))
    out = lax.dot_general(selb, acc_ref[...],
                          dimension_numbers=(((2,), (1,)), ((0,), (0,))),
                          preferred_element_type=jnp.float32)
    o_ref[0] = out.astype(o_ref.dtype)


def _conv3x3_s2(x, w9, b, apply_relu=False):
    N, H_in, W_in, Cin = x.shape
    Cout = w9.shape[-1]
    H2 = H_in // 2
    W_out = (W_in - 1) // 2 + 1
    body = functools.partial(_c3s2_kernel, H2=H2, W_in=W_in, W_out=W_out,
                             Cout=Cout, apply_relu=apply_relu)
    return pl.pallas_call(
        body,
        out_shape=jax.ShapeDtypeStruct((N, H2, W_out, Cout), jnp.float32),
        grid=(N,),
        in_specs=[
            pl.BlockSpec((1, H_in, W_in, Cin), lambda n: (n, 0, 0, 0)),
            pl.BlockSpec((9, Cin, Cout), lambda n: (0, 0, 0)),
            pl.BlockSpec((1, Cout), lambda n: (0, 0)),
        ],
        out_specs=pl.BlockSpec((1, H2, W_out, Cout), lambda n: (n, 0, 0, 0)),
        scratch_shapes=[pltpu.VMEM((H2, W_in, Cout), jnp.float32)],
        compiler_params=pltpu.CompilerParams(
            dimension_semantics=("parallel",),
            vmem_limit_bytes=_VMEM_LIMIT),
    )(x, w9, b.reshape(1, Cout))


def kernel(C3, C4, C5, P5_1_w, P5_1_b, P5_2_w, P5_2_b, P4_1_w, P4_1_b,
           P4_2_w, P4_2_b, P3_1_w, P3_1_b, P3_2_w, P3_2_b, P6_w, P6_b,
           P7_2_w, P7_2_b):
    to_nhwc = lambda t: jnp.transpose(t.astype(_BF), (0, 2, 3, 1))
    c3 = to_nhwc(C3)
    c4 = to_nhwc(C4)
    c5 = to_nhwc(C5)

    p5_lat = _conv1x1(c5, P5_1_w.astype(_BF), P5_1_b)
    p5 = _conv3x3_s1(p5_lat.astype(_BF), P5_2_w.astype(_BF), P5_2_b, 20)

    p4_lat = _conv1x1_upsample_add(c4, P4_1_w.astype(_BF), P4_1_b,
                                   p5_lat.astype(_BF))
    p4 = _conv3x3_s1(p4_lat.astype(_BF), P4_2_w.astype(_BF), P4_2_b, 20)

    p3_lat = _conv1x1_upsample_add(c3, P3_1_w.astype(_BF), P3_1_b,
                                   p4_lat.astype(_BF))
    p3 = _conv3x3_s1(p3_lat.astype(_BF), P3_2_w.astype(_BF), P3_2_b, 20)

    p6 = _conv3x3_s2(c5, P6_w.astype(_BF), P6_b)
    p7 = _conv3x3_s2(p6.astype(_BF), P7_2_w.astype(_BF), P7_2_b,
                     apply_relu=True)

    to_nchw = lambda t: jnp.transpose(t, (0, 3, 1, 2))
    return [to_nchw(p3), to_nchw(p4), to_nchw(p5), to_nchw(p6), to_nchw(p7)]
